# Initial kernel scaffold; baseline (speedup 1.0000x reference)
import jax
import jax.numpy as jnp
from jax.experimental import pallas as pl


def kernel(token_ids, embedding):
    raise NotImplementedError("write your pallas kernel here")


# speedup vs baseline: 5.5733x; 5.5733x over previous
import functools

import jax
import jax.numpy as jnp
from jax import lax
from jax.experimental import pallas as pl
from jax.experimental.pallas import tpu as pltpu
from jax.experimental.pallas import tpu_sc as plsc

NC = 2
NS = 16
NW = NC * NS
D = 64
CH = 512
NBUF = 2


def _gather_kernel(B):
    bw = B // NW
    n_ch = bw // CH
    assert bw % CH == 0 and n_ch % NBUF == 0
    mesh = plsc.VectorSubcoreMesh(
        core_axis_name="c", subcore_axis_name="s",
        num_cores=NC, num_subcores=NS)

    @functools.partial(
        pl.kernel,
        out_type=jax.ShapeDtypeStruct((B, D), jnp.float32),
        mesh=mesh,
        compiler_params=pltpu.CompilerParams(use_tc_tiling_on_sc=False),
        scratch_types=[
            pltpu.VMEM((CH,), jnp.int32),
            pltpu.VMEM((CH,), jnp.int32),
            pltpu.VMEM((CH, D), jnp.float32),
            pltpu.VMEM((CH, D), jnp.float32),
            pltpu.SemaphoreType.DMA,
            pltpu.SemaphoreType.DMA,
        ],
    )
    def body(idx_hbm, table_hbm, out_hbm, idx0, idx1, rows0, rows1,
             sem0, sem1):
        idxs = (idx0, idx1)
        rows = (rows0, rows1)
        sems = (sem0, sem1)
        wid = lax.axis_index("s") * NC + lax.axis_index("c")
        base = wid * bw

        for b in range(NBUF):
            pltpu.sync_copy(idx_hbm.at[pl.ds(base + b * CH, CH)], idxs[b])
            pltpu.async_copy(table_hbm.at[idxs[b]], rows[b], sems[b])

        def pair(g, _):
            for b in range(NBUF):
                i = g * NBUF + b
                pltpu.make_async_copy(
                    table_hbm.at[idxs[b]], rows[b], sems[b]).wait()
                pltpu.sync_copy(
                    rows[b], out_hbm.at[pl.ds(base + i * CH, CH)])

                @pl.when(i + NBUF < n_ch)
                def _():
                    pltpu.sync_copy(
                        idx_hbm.at[pl.ds(base + (i + NBUF) * CH, CH)],
                        idxs[b])
                    pltpu.async_copy(
                        table_hbm.at[idxs[b]], rows[b], sems[b])
            return _

        lax.fori_loop(0, n_ch // NBUF, pair, None)

    return body


def kernel(token_ids, embedding):
    B = token_ids.size
    idx = jnp.reshape(token_ids, (B,)).astype(jnp.int32)
    out = _gather_kernel(B)(idx, embedding)
    return jnp.reshape(out, token_ids.shape + (D,))
